# Initial kernel scaffold; baseline (speedup 1.0000x reference)
import jax
import jax.numpy as jnp
from jax.experimental import pallas as pl


def kernel(flat, cu_seqlens):
    raise NotImplementedError("write your pallas kernel here")


# speedup vs baseline: 4.5842x; 4.5842x over previous
import dataclasses

import jax
import jax.numpy as jnp
from jax import lax
from jax.experimental import pallas as pl
from jax.experimental.pallas import tpu as pltpu
from jax.experimental.pallas import tpu_sc as plsc

_CP = pltpu.CompilerParams()
if "needs_layout_passes" in pltpu.CompilerParams.__dataclass_fields__:
    _CP = dataclasses.replace(_CP, needs_layout_passes=False)

_B = 16
_MAX_LEN = 4096
_D = 512
_TOTAL = 32768

_P = 64
_NW = 32
_NPIECES = (_B * _MAX_LEN) // _P
_PER_W = _NPIECES // _NW


def _pad_and_stack_sc(flat, cu16):
    mesh = plsc.VectorSubcoreMesh(core_axis_name="c", subcore_axis_name="s")

    @pl.kernel(
        out_type=jax.ShapeDtypeStruct((_B * _MAX_LEN, _D), jnp.float32),
        mesh=mesh,
        compiler_params=_CP,
        scratch_types=[
            pltpu.VMEM((_P, _D), jnp.float32),
            pltpu.VMEM((_P, _D), jnp.float32),
            pltpu.VMEM((_P,), jnp.int32),
            pltpu.VMEM((16,), jnp.int32),
        ],
    )
    def k(flat_hbm, cu_hbm, out_hbm, buf, zbuf, idx_v, cu_v):
        wid = lax.axis_index("s") * 2 + lax.axis_index("c")

        @pl.loop(0, _P)
        def _zero_row(r):
            for j in range(_D // 16):
                zbuf[r, pl.ds(j * 16, 16)] = jnp.zeros((16,), jnp.float32)

        pltpu.sync_copy(cu_hbm.at[pl.ds(0, 16)], cu_v)
        cuvec = cu_v[...]
        lane = lax.iota(jnp.int32, 16)

        def cu_at(i):
            v = jnp.sum(jnp.where(lane == i, cuvec, 0))
            return jnp.where(i >= _B, _TOTAL, v)

        @pl.loop(0, _PER_W)
        def _piece(i):
            pidx = i * _NW + wid
            row0 = pidx * _P
            b = row0 // _MAX_LEN
            m0 = row0 % _MAX_LEN
            cu_b = cu_at(b)
            cu_b1 = cu_at(b + 1)
            nv = jnp.clip(cu_b1 - cu_b - m0, 0, _P)
            src = cu_b + m0

            @pl.when(nv == 0)
            def _all_pad():
                pltpu.sync_copy(zbuf, out_hbm.at[pl.ds(row0, _P)])

            @pl.when(nv > 0)
            def _data():
                for q in range(_P // 16):
                    idx_v[pl.ds(q * 16, 16)] = jnp.minimum(
                        src + lane + (q * 16), _TOTAL - 1)
                pltpu.sync_copy(flat_hbm.at[idx_v], buf)

                @pl.loop(nv, _P)
                def _zero_tail(r):
                    for j in range(_D // 16):
                        buf[r, pl.ds(j * 16, 16)] = jnp.zeros(
                            (16,), jnp.float32)

                pltpu.sync_copy(buf, out_hbm.at[pl.ds(row0, _P)])

    return k(flat, cu16)


@jax.jit
def kernel(flat, cu_seqlens):
    cu16 = cu_seqlens[:16]
    out = _pad_and_stack_sc(flat, cu16)
    return out.reshape(_B, _MAX_LEN, _D)


# speedup vs baseline: 5.0256x; 1.0963x over previous
import dataclasses

import jax
import jax.numpy as jnp
from jax import lax
from jax.experimental import pallas as pl
from jax.experimental.pallas import tpu as pltpu
from jax.experimental.pallas import tpu_sc as plsc

_CP = pltpu.CompilerParams()
if "needs_layout_passes" in pltpu.CompilerParams.__dataclass_fields__:
    _CP = dataclasses.replace(_CP, needs_layout_passes=False)

_B = 16
_MAX_LEN = 4096
_D = 512
_TOTAL = 32768

_P = 64
_NW = 32
_NPIECES = (_B * _MAX_LEN) // _P
_PER_W = _NPIECES // _NW


def _pad_and_stack_sc(flat, cu16):
    mesh = plsc.VectorSubcoreMesh(core_axis_name="c", subcore_axis_name="s")

    @pl.kernel(
        out_type=jax.ShapeDtypeStruct((_B * _MAX_LEN, _D), jnp.float32),
        mesh=mesh,
        compiler_params=_CP,
        scratch_types=[
            pltpu.VMEM((_P, _D), jnp.float32),
            pltpu.VMEM((_P, _D), jnp.float32),
            pltpu.VMEM((_P, _D), jnp.float32),
            pltpu.VMEM((_P,), jnp.int32),
            pltpu.VMEM((16,), jnp.int32),
            pltpu.SemaphoreType.DMA,
            pltpu.SemaphoreType.DMA,
            pltpu.SemaphoreType.DMA,
        ],
    )
    def k(flat_hbm, cu_hbm, out_hbm, buf0, buf1, zbuf, idx_v, cu_v,
          in_sem, out_sem0, out_sem1):
        bufs = (buf0, buf1)
        out_sems = (out_sem0, out_sem1)
        wid = lax.axis_index("s") * 2 + lax.axis_index("c")

        @pl.loop(0, _P)
        def _zero_row(r):
            for j in range(_D // 16):
                zbuf[r, pl.ds(j * 16, 16)] = jnp.zeros((16,), jnp.float32)

        pltpu.sync_copy(cu_hbm.at[pl.ds(0, 16)], cu_v)
        cuvec = cu_v[...]
        lane = lax.iota(jnp.int32, 16)

        def cu_at(i):
            v = jnp.sum(jnp.where(lane == i, cuvec, 0))
            return jnp.where(i >= _B, _TOTAL, v)

        def do_piece(i, par):
            slot = bufs[par]
            pidx = i * _NW + wid
            row0 = pidx * _P
            b = row0 // _MAX_LEN
            m0 = row0 % _MAX_LEN
            cu_b = cu_at(b)
            cu_b1 = cu_at(b + 1)
            nv = jnp.clip(cu_b1 - cu_b - m0, 0, _P)
            src = cu_b + m0

            @pl.when(i >= 2)
            def _drain_slot():
                pltpu.make_async_copy(
                    zbuf, out_hbm.at[pl.ds(row0, _P)], out_sems[par]).wait()

            @pl.when(nv == 0)
            def _all_pad():
                pltpu.make_async_copy(
                    zbuf, out_hbm.at[pl.ds(row0, _P)], out_sems[par]).start()

            @pl.when(nv > 0)
            def _data():
                for q in range(_P // 16):
                    idx_v[pl.ds(q * 16, 16)] = jnp.minimum(
                        src + lane + (q * 16), _TOTAL - 1)
                pltpu.async_copy(flat_hbm.at[idx_v], slot, in_sem).wait()

                @pl.loop(nv, _P)
                def _zero_tail(r):
                    for j in range(_D // 16):
                        slot[r, pl.ds(j * 16, 16)] = jnp.zeros(
                            (16,), jnp.float32)

                pltpu.make_async_copy(
                    slot, out_hbm.at[pl.ds(row0, _P)], out_sems[par]).start()

        @pl.loop(0, _PER_W, step=2)
        def _piece(i):
            do_piece(i, 0)
            do_piece(i + 1, 1)

        for par in range(2):
            pltpu.make_async_copy(
                zbuf, out_hbm.at[pl.ds(0, _P)], out_sems[par]).wait()

    return k(flat, cu16)


@jax.jit
def kernel(flat, cu_seqlens):
    cu16 = cu_seqlens[:16]
    out = _pad_and_stack_sc(flat, cu16)
    return out.reshape(_B, _MAX_LEN, _D)
